# baseline (speedup 1.0000x reference)
import functools

import jax
import jax.numpy as jnp
from jax import lax
from jax.experimental import pallas as pl
from jax.experimental.pallas import tpu as pltpu
from jax.experimental.pallas import tpu_sc as plsc

VOCAB = 1000000
EMB = 64
CLASSES = 100
BATCH = 4096
SENT_COUNT = 26
SENT_LEN = 20

NUM_BAGS = BATCH * SENT_COUNT
NW = 32
BAGS_PER_W = NUM_BAGS // NW
BAGS_PER_GATHER = 4
IDX_PER_GATHER = BAGS_PER_GATHER * SENT_LEN
GATHERS_PER_W = BAGS_PER_W // BAGS_PER_GATHER
GATHERS_PER_SC = 104
SUPER_PER_W = GATHERS_PER_W // GATHERS_PER_SC
BAGS_PER_SC = GATHERS_PER_SC * BAGS_PER_GATHER
NBUF = 4
TOK_ROWS = NUM_BAGS * SENT_LEN // IDX_PER_GATHER

CPAD = 128
BB = 512
QS = EMB // 16


def _embbag_sc(emb_weight, tok2d):
    mesh = plsc.VectorSubcoreMesh(core_axis_name="c", subcore_axis_name="s",
                                  num_cores=2, num_subcores=16)

    @functools.partial(
        pl.kernel,
        out_type=jax.ShapeDtypeStruct((NUM_BAGS, EMB), jnp.float32),
        mesh=mesh,
        scratch_types=[
            pltpu.VMEM((2, GATHERS_PER_SC, IDX_PER_GATHER), jnp.int32),
            pltpu.VMEM((NBUF, IDX_PER_GATHER, EMB), jnp.float32),
            pltpu.VMEM((2, BAGS_PER_SC, EMB), jnp.float32),
            [pltpu.SemaphoreType.DMA] * NBUF,
            pltpu.SemaphoreType.DMA,
            pltpu.SemaphoreType.DMA,
        ],
        compiler_params=pltpu.CompilerParams(use_tc_tiling_on_sc=False),
    )
    def k(emb_hbm, tok_hbm, out_hbm, idx_v, rows_v, sents_v, sem_g,
          sem_idx, sem_out):
        wid = lax.axis_index("s") * 2 + lax.axis_index("c")

        def idx_rows(sc):
            return wid * GATHERS_PER_W + sc * GATHERS_PER_SC

        def to_packed_rows(ib, row):
            for q in range(IDX_PER_GATHER // 16):
                t = idx_v[ib, row, pl.ds(q * 16, 16)]
                r = ((t & (-2 * THALF)) | ((t & (THALF - 1)) << 1)
                     | ((t >> 12) & 1))
                idx_v[ib, row, pl.ds(q * 16, 16)] = r

        def start_gather(ib, j, b):
            pltpu.async_copy(emb_hbm.at[idx_v.at[ib, j]], rows_v.at[b],
                             sem_g[b])

        def wait_gather(b):
            pltpu.make_async_copy(emb_hbm.at[idx_v.at[0, 0]], rows_v.at[b],
                                  sem_g[b]).wait()

        def accumulate(ob, j, b):
            for bag in range(BAGS_PER_GATHER):
                row = j * BAGS_PER_GATHER + bag
                for q in range(QS):
                    vals = [rows_v[b, SENT_LEN * bag + t, pl.ds(q * 16, 16)]
                            for t in range(SENT_LEN)]
                    while len(vals) > 1:
                        nxt = [vals[i] + vals[i + 1]
                               for i in range(0, len(vals) - 1, 2)]
                        if len(vals) % 2:
                            nxt[-1] = nxt[-1] + vals[-1]
                        vals = nxt
                    sents_v[ob, row, pl.ds(q * 16, 16)] = \
                        vals[0] * (1.0 / SENT_LEN)

        def prime(ib):
            for b in range(NBUF):
                to_packed_rows(ib, b)
                start_gather(ib, b, b)

        pltpu.sync_copy(tok_hbm.at[pl.ds(idx_rows(0), GATHERS_PER_SC)],
                        idx_v.at[0])
        pltpu.async_copy(tok_hbm.at[pl.ds(idx_rows(1), GATHERS_PER_SC)],
                         idx_v.at[1], sem_idx)
        prime(0)

        def super_chunk(sc, carry):
            ib = sc & 1
            ob = sc & 1

            @pl.when(sc >= 2)
            def _():
                pltpu.make_async_copy(
                    sents_v.at[ob],
                    out_hbm.at[pl.ds(wid * BAGS_PER_W, BAGS_PER_SC)],
                    sem_out).wait()

            def quad(g, c):
                for b in range(NBUF):
                    j = g * NBUF + b
                    wait_gather(b)
                    accumulate(ob, j, b)

                    @pl.when(j + NBUF < GATHERS_PER_SC)
                    def _():
                        to_packed_rows(ib, j + NBUF)
                        start_gather(ib, j + NBUF, b)
                return c

            lax.fori_loop(0, GATHERS_PER_SC // NBUF, quad, 0)

            bag0 = wid * BAGS_PER_W + sc * BAGS_PER_SC
            pltpu.async_copy(sents_v.at[ob],
                             out_hbm.at[pl.ds(bag0, BAGS_PER_SC)], sem_out)

            @pl.when(sc + 1 < SUPER_PER_W)
            def _():
                pltpu.make_async_copy(
                    tok_hbm.at[pl.ds(idx_rows(0), GATHERS_PER_SC)],
                    idx_v.at[1 - ib], sem_idx).wait()
                prime(1 - ib)

                @pl.when(sc + 2 < SUPER_PER_W)
                def _():
                    pltpu.async_copy(
                        tok_hbm.at[pl.ds(idx_rows(sc + 2), GATHERS_PER_SC)],
                        idx_v.at[ib], sem_idx)
            return carry

        lax.fori_loop(0, SUPER_PER_W, super_chunk, 0)
        for _ in range(2):
            pltpu.make_async_copy(
                sents_v.at[0],
                out_hbm.at[pl.ds(wid * BAGS_PER_W, BAGS_PER_SC)],
                sem_out).wait()

    return k(emb_weight, tok2d)


THALF = 4096
TCHUNKS = (VOCAB + 2 * THALF - 1) // (2 * THALF)
PACK_ROWS = TCHUNKS * THALF
LIN_ROWS = 2 * PACK_ROWS


def _transpose_kernel(a_ref, b_ref, out_ref):
    out_ref[:, 0:EMB] = a_ref[...].T
    out_ref[:, EMB:2 * EMB] = b_ref[...].T


def _relayout_table_tc(emb_t):
    out = pl.pallas_call(
        _transpose_kernel,
        grid=(TCHUNKS,),
        in_specs=[
            pl.BlockSpec((EMB, THALF), lambda i: (0, 2 * i)),
            pl.BlockSpec((EMB, THALF),
                         lambda i: (0, jnp.minimum(2 * i + 1, 2 * TCHUNKS - 2))),
        ],
        out_specs=pl.BlockSpec((THALF, 2 * EMB), lambda i: (i, 0)),
        out_shape=jax.ShapeDtypeStruct((PACK_ROWS, 2 * EMB), jnp.float32),
        compiler_params=pltpu.CompilerParams(
            dimension_semantics=("arbitrary",)),
    )(emb_t, emb_t)
    return out.reshape(LIN_ROWS, EMB)


def _attention_kernel(sents_ref, cls_ref, mul_ref, bias_ref, out_ref, sc_ref):
    m = jnp.dot(sents_ref[0], cls_ref[...], preferred_element_type=jnp.float32)
    sc_ref[0] = m
    for s in range(1, SENT_COUNT):
        scores = jnp.dot(sents_ref[s], cls_ref[...],
                         preferred_element_type=jnp.float32)
        sc_ref[s] = scores
        m = jnp.maximum(m, scores)
    den = jnp.zeros((BB, CPAD), jnp.float32)
    num = jnp.zeros((BB, CPAD), jnp.float32)
    for s in range(SENT_COUNT):
        e = jnp.exp(sc_ref[s] - m)
        den = den + e
        proj = jnp.dot(sents_ref[s], mul_ref[...],
                       preferred_element_type=jnp.float32)
        num = num + e * proj
    out_ref[...] = num / den + bias_ref[...]


def _attention_tc(sents_t, class_embs, multi_weight, multi_bias):
    cls_p = jnp.zeros((EMB, CPAD), jnp.float32).at[:, :CLASSES].set(class_embs.T)
    mul_p = jnp.zeros((EMB, CPAD), jnp.float32).at[:, :CLASSES].set(multi_weight.T)
    bias_p = jnp.zeros((1, CPAD), jnp.float32).at[0, :CLASSES].set(multi_bias)
    grid = BATCH // BB
    out = pl.pallas_call(
        _attention_kernel,
        grid=(grid,),
        in_specs=[
            pl.BlockSpec((SENT_COUNT, BB, EMB), lambda i: (0, i, 0)),
            pl.BlockSpec((EMB, CPAD), lambda i: (0, 0)),
            pl.BlockSpec((EMB, CPAD), lambda i: (0, 0)),
            pl.BlockSpec((1, CPAD), lambda i: (0, 0)),
        ],
        out_specs=pl.BlockSpec((BB, CPAD), lambda i: (i, 0)),
        out_shape=jax.ShapeDtypeStruct((BATCH, CPAD), jnp.float32),
        scratch_shapes=[pltpu.VMEM((SENT_COUNT, BB, CPAD), jnp.float32)],
        compiler_params=pltpu.CompilerParams(
            dimension_semantics=("arbitrary",)),
    )(sents_t, cls_p, mul_p, bias_p)
    return out[:, :CLASSES]


def kernel(tok_lists_batch, emb_weight, class_embs, multi_weight, multi_bias):
    emb_lin = _relayout_table_tc(emb_weight.T)
    assert emb_lin.shape == (LIN_ROWS, EMB)
    tok2d = tok_lists_batch.transpose(1, 0, 2).reshape(TOK_ROWS, IDX_PER_GATHER)
    sents_flat = _embbag_sc(emb_lin, tok2d)
    sents_t = sents_flat.reshape(SENT_COUNT, BATCH, EMB)
    return _attention_tc(sents_t, class_embs, multi_weight, multi_bias)


# speedup vs baseline: 1.1446x; 1.1446x over previous
import functools

import jax
import jax.numpy as jnp
from jax import lax
from jax.experimental import pallas as pl
from jax.experimental.pallas import tpu as pltpu
from jax.experimental.pallas import tpu_sc as plsc

VOCAB = 1000000
EMB = 64
CLASSES = 100
BATCH = 4096
SENT_COUNT = 26
SENT_LEN = 20

NUM_BAGS = BATCH * SENT_COUNT
NW = 32
BAGS_PER_W = NUM_BAGS // NW
BAGS_PER_GATHER = 4
IDX_PER_GATHER = BAGS_PER_GATHER * SENT_LEN
GATHERS_PER_W = BAGS_PER_W // BAGS_PER_GATHER
GATHERS_PER_SC = 104
SUPER_PER_W = GATHERS_PER_W // GATHERS_PER_SC
BAGS_PER_SC = GATHERS_PER_SC * BAGS_PER_GATHER
NBUF = 4
TOK_ROWS = NUM_BAGS * SENT_LEN // IDX_PER_GATHER

CPAD = 128
BB = 512
QS = EMB // 16


def _embbag_sc(emb_weight, tok2d):
    mesh = plsc.VectorSubcoreMesh(core_axis_name="c", subcore_axis_name="s",
                                  num_cores=2, num_subcores=16)

    @functools.partial(
        pl.kernel,
        out_type=jax.ShapeDtypeStruct((NUM_BAGS, EMB), jnp.float32),
        mesh=mesh,
        scratch_types=[
            pltpu.VMEM((2, GATHERS_PER_SC, IDX_PER_GATHER), jnp.int32),
            pltpu.VMEM((NBUF, IDX_PER_GATHER, EMB), jnp.float32),
            pltpu.VMEM((2, BAGS_PER_SC, EMB), jnp.float32),
            [pltpu.SemaphoreType.DMA] * NBUF,
            pltpu.SemaphoreType.DMA,
            pltpu.SemaphoreType.DMA,
        ],
        compiler_params=pltpu.CompilerParams(use_tc_tiling_on_sc=False),
    )
    def k(emb_hbm, tok_hbm, out_hbm, idx_v, rows_v, sents_v, sem_g,
          sem_idx, sem_out):
        wid = lax.axis_index("s") * 2 + lax.axis_index("c")

        def idx_rows(sc):
            return wid * GATHERS_PER_W + sc * GATHERS_PER_SC

        def to_packed_rows(ib, row):
            for q in range(IDX_PER_GATHER // 16):
                t = idx_v[ib, row, pl.ds(q * 16, 16)]
                r = ((t & (-2 * THALF)) | ((t & (THALF - 1)) << 1)
                     | ((t >> 12) & 1))
                idx_v[ib, row, pl.ds(q * 16, 16)] = r

        def start_gather(ib, j, b):
            pltpu.async_copy(emb_hbm.at[idx_v.at[ib, j]], rows_v.at[b],
                             sem_g[b])

        def wait_gather(b):
            pltpu.make_async_copy(emb_hbm.at[idx_v.at[0, 0]], rows_v.at[b],
                                  sem_g[b]).wait()

        def accumulate(ob, j, b):
            for bag in range(BAGS_PER_GATHER):
                row = j * BAGS_PER_GATHER + bag
                base = SENT_LEN * bag
                for q in range(QS):
                    a0 = rows_v[b, base, pl.ds(q * 16, 16)]
                    a1 = rows_v[b, base + 1, pl.ds(q * 16, 16)]
                    for t in range(2, SENT_LEN, 2):
                        a0 = a0 + rows_v[b, base + t, pl.ds(q * 16, 16)]
                        a1 = a1 + rows_v[b, base + t + 1, pl.ds(q * 16, 16)]
                    sents_v[ob, row, pl.ds(q * 16, 16)] = \
                        (a0 + a1) * (1.0 / SENT_LEN)

        def prime(ib):
            for b in range(NBUF):
                to_packed_rows(ib, b)
                start_gather(ib, b, b)

        pltpu.sync_copy(tok_hbm.at[pl.ds(idx_rows(0), GATHERS_PER_SC)],
                        idx_v.at[0])
        pltpu.async_copy(tok_hbm.at[pl.ds(idx_rows(1), GATHERS_PER_SC)],
                         idx_v.at[1], sem_idx)
        prime(0)

        def super_chunk(sc, carry):
            ib = sc & 1
            ob = sc & 1

            @pl.when(sc >= 2)
            def _():
                pltpu.make_async_copy(
                    sents_v.at[ob],
                    out_hbm.at[pl.ds(wid * BAGS_PER_W, BAGS_PER_SC)],
                    sem_out).wait()

            def quad(g, c):
                for b in range(NBUF):
                    j = g * NBUF + b
                    wait_gather(b)
                    accumulate(ob, j, b)

                    @pl.when(j + NBUF < GATHERS_PER_SC)
                    def _():
                        to_packed_rows(ib, j + NBUF)
                        start_gather(ib, j + NBUF, b)
                return c

            lax.fori_loop(0, GATHERS_PER_SC // NBUF, quad, 0)

            bag0 = wid * BAGS_PER_W + sc * BAGS_PER_SC
            pltpu.async_copy(sents_v.at[ob],
                             out_hbm.at[pl.ds(bag0, BAGS_PER_SC)], sem_out)

            @pl.when(sc + 1 < SUPER_PER_W)
            def _():
                pltpu.make_async_copy(
                    tok_hbm.at[pl.ds(idx_rows(0), GATHERS_PER_SC)],
                    idx_v.at[1 - ib], sem_idx).wait()
                prime(1 - ib)

                @pl.when(sc + 2 < SUPER_PER_W)
                def _():
                    pltpu.async_copy(
                        tok_hbm.at[pl.ds(idx_rows(sc + 2), GATHERS_PER_SC)],
                        idx_v.at[ib], sem_idx)
            return carry

        lax.fori_loop(0, SUPER_PER_W, super_chunk, 0)
        for _ in range(2):
            pltpu.make_async_copy(
                sents_v.at[0],
                out_hbm.at[pl.ds(wid * BAGS_PER_W, BAGS_PER_SC)],
                sem_out).wait()

    return k(emb_weight, tok2d)


THALF = 4096
TCHUNKS = (VOCAB + 2 * THALF - 1) // (2 * THALF)
PACK_ROWS = TCHUNKS * THALF
LIN_ROWS = 2 * PACK_ROWS


def _transpose_kernel(a_ref, b_ref, out_ref):
    out_ref[:, 0:EMB] = a_ref[...].T
    out_ref[:, EMB:2 * EMB] = b_ref[...].T


def _relayout_table_tc(emb_t):
    out = pl.pallas_call(
        _transpose_kernel,
        grid=(TCHUNKS,),
        in_specs=[
            pl.BlockSpec((EMB, THALF), lambda i: (0, 2 * i)),
            pl.BlockSpec((EMB, THALF),
                         lambda i: (0, jnp.minimum(2 * i + 1, 2 * TCHUNKS - 2))),
        ],
        out_specs=pl.BlockSpec((THALF, 2 * EMB), lambda i: (i, 0)),
        out_shape=jax.ShapeDtypeStruct((PACK_ROWS, 2 * EMB), jnp.float32),
        compiler_params=pltpu.CompilerParams(
            dimension_semantics=("arbitrary",)),
    )(emb_t, emb_t)
    return out.reshape(LIN_ROWS, EMB)


def _attention_kernel(sents_ref, cls_ref, mul_ref, bias_ref, out_ref, sc_ref):
    m = jnp.dot(sents_ref[0], cls_ref[...], preferred_element_type=jnp.float32)
    sc_ref[0] = m
    for s in range(1, SENT_COUNT):
        scores = jnp.dot(sents_ref[s], cls_ref[...],
                         preferred_element_type=jnp.float32)
        sc_ref[s] = scores
        m = jnp.maximum(m, scores)
    den = jnp.zeros((BB, CPAD), jnp.float32)
    num = jnp.zeros((BB, CPAD), jnp.float32)
    for s in range(SENT_COUNT):
        e = jnp.exp(sc_ref[s] - m)
        den = den + e
        proj = jnp.dot(sents_ref[s], mul_ref[...],
                       preferred_element_type=jnp.float32)
        num = num + e * proj
    out_ref[...] = num / den + bias_ref[...]


def _attention_tc(sents_t, class_embs, multi_weight, multi_bias):
    cls_p = jnp.zeros((EMB, CPAD), jnp.float32).at[:, :CLASSES].set(class_embs.T)
    mul_p = jnp.zeros((EMB, CPAD), jnp.float32).at[:, :CLASSES].set(multi_weight.T)
    bias_p = jnp.zeros((1, CPAD), jnp.float32).at[0, :CLASSES].set(multi_bias)
    grid = BATCH // BB
    out = pl.pallas_call(
        _attention_kernel,
        grid=(grid,),
        in_specs=[
            pl.BlockSpec((SENT_COUNT, BB, EMB), lambda i: (0, i, 0)),
            pl.BlockSpec((EMB, CPAD), lambda i: (0, 0)),
            pl.BlockSpec((EMB, CPAD), lambda i: (0, 0)),
            pl.BlockSpec((1, CPAD), lambda i: (0, 0)),
        ],
        out_specs=pl.BlockSpec((BB, CPAD), lambda i: (i, 0)),
        out_shape=jax.ShapeDtypeStruct((BATCH, CPAD), jnp.float32),
        scratch_shapes=[pltpu.VMEM((SENT_COUNT, BB, CPAD), jnp.float32)],
        compiler_params=pltpu.CompilerParams(
            dimension_semantics=("arbitrary",)),
    )(sents_t, cls_p, mul_p, bias_p)
    return out[:, :CLASSES]


def kernel(tok_lists_batch, emb_weight, class_embs, multi_weight, multi_bias):
    emb_lin = _relayout_table_tc(emb_weight.T)
    assert emb_lin.shape == (LIN_ROWS, EMB)
    tok2d = tok_lists_batch.transpose(1, 0, 2).reshape(TOK_ROWS, IDX_PER_GATHER)
    sents_flat = _embbag_sc(emb_lin, tok2d)
    sents_t = sents_flat.reshape(SENT_COUNT, BATCH, EMB)
    return _attention_tc(sents_t, class_embs, multi_weight, multi_bias)


# speedup vs baseline: 1.2729x; 1.1121x over previous
import functools

import jax
import jax.numpy as jnp
from jax import lax
from jax.experimental import pallas as pl
from jax.experimental.pallas import tpu as pltpu
from jax.experimental.pallas import tpu_sc as plsc

VOCAB = 1000000
EMB = 64
CLASSES = 100
BATCH = 4096
SENT_COUNT = 26
SENT_LEN = 20

NUM_BAGS = BATCH * SENT_COUNT
NW = 32
BAGS_PER_W = NUM_BAGS // NW
BAGS_PER_GATHER = 4
IDX_PER_GATHER = BAGS_PER_GATHER * SENT_LEN
GATHERS_PER_W = BAGS_PER_W // BAGS_PER_GATHER
GATHERS_PER_SC = 104
SUPER_PER_W = GATHERS_PER_W // GATHERS_PER_SC
BAGS_PER_SC = GATHERS_PER_SC * BAGS_PER_GATHER
NBUF = 4
TOK_ROWS = NUM_BAGS * SENT_LEN // IDX_PER_GATHER

CPAD = 128
BB = 512
QS = EMB // 16


def _embbag_sc(emb_weight, tok2d):
    mesh = plsc.VectorSubcoreMesh(core_axis_name="c", subcore_axis_name="s",
                                  num_cores=2, num_subcores=16)

    @functools.partial(
        pl.kernel,
        out_type=jax.ShapeDtypeStruct((NUM_BAGS, EMB), jnp.float32),
        mesh=mesh,
        scratch_types=[
            pltpu.VMEM((2, GATHERS_PER_SC, IDX_PER_GATHER), jnp.int32),
            pltpu.VMEM((NBUF, IDX_PER_GATHER, EMB), jnp.float32),
            pltpu.VMEM((2, BAGS_PER_SC, EMB), jnp.float32),
            [pltpu.SemaphoreType.DMA] * NBUF,
            pltpu.SemaphoreType.DMA,
            pltpu.SemaphoreType.DMA,
        ],
        compiler_params=pltpu.CompilerParams(use_tc_tiling_on_sc=False),
    )
    def k(emb_hbm, tok_hbm, out_hbm, idx_v, rows_v, sents_v, sem_g,
          sem_idx, sem_out):
        wid = lax.axis_index("s") * 2 + lax.axis_index("c")

        def idx_rows(sc):
            return wid * GATHERS_PER_W + sc * GATHERS_PER_SC

        def to_packed_rows(ib, row):
            for q in range(IDX_PER_GATHER // 16):
                t = idx_v[ib, row, pl.ds(q * 16, 16)]
                r = ((t & (-2 * THALF)) | ((t & (THALF - 1)) << 1)
                     | ((t >> 12) & 1))
                idx_v[ib, row, pl.ds(q * 16, 16)] = r

        def start_gather(ib, j, b):
            pltpu.async_copy(emb_hbm.at[idx_v.at[ib, j]], rows_v.at[b],
                             sem_g[b])

        def wait_gather(b):
            pltpu.make_async_copy(emb_hbm.at[idx_v.at[0, 0]], rows_v.at[b],
                                  sem_g[b]).wait()

        def accumulate(ob, j, b):
            for bag in range(BAGS_PER_GATHER):
                acc = [rows_v[b, SENT_LEN * bag, pl.ds(q * 16, 16)]
                       for q in range(QS)]
                for t in range(1, SENT_LEN):
                    for q in range(QS):
                        acc[q] = acc[q] + rows_v[b, SENT_LEN * bag + t,
                                                 pl.ds(q * 16, 16)]
                row = j * BAGS_PER_GATHER + bag
                for q in range(QS):
                    sents_v[ob, row, pl.ds(q * 16, 16)] = \
                        acc[q] * (1.0 / SENT_LEN)

        def prime(ib):
            for b in range(NBUF):
                to_packed_rows(ib, b)
                start_gather(ib, b, b)

        pltpu.sync_copy(tok_hbm.at[pl.ds(idx_rows(0), GATHERS_PER_SC)],
                        idx_v.at[0])
        pltpu.async_copy(tok_hbm.at[pl.ds(idx_rows(1), GATHERS_PER_SC)],
                         idx_v.at[1], sem_idx)
        prime(0)

        def super_chunk(sc, carry):
            ib = sc & 1
            ob = sc & 1

            @pl.when(sc >= 2)
            def _():
                pltpu.make_async_copy(
                    sents_v.at[ob],
                    out_hbm.at[pl.ds(wid * BAGS_PER_W, BAGS_PER_SC)],
                    sem_out).wait()

            def quad(g, c):
                for b in range(NBUF):
                    j = g * NBUF + b
                    wait_gather(b)
                    accumulate(ob, j, b)

                    @pl.when(j + NBUF < GATHERS_PER_SC)
                    def _():
                        to_packed_rows(ib, j + NBUF)
                        start_gather(ib, j + NBUF, b)
                return c

            lax.fori_loop(0, GATHERS_PER_SC // NBUF, quad, 0)

            bag0 = wid * BAGS_PER_W + sc * BAGS_PER_SC
            pltpu.async_copy(sents_v.at[ob],
                             out_hbm.at[pl.ds(bag0, BAGS_PER_SC)], sem_out)

            @pl.when(sc + 1 < SUPER_PER_W)
            def _():
                pltpu.make_async_copy(
                    tok_hbm.at[pl.ds(idx_rows(0), GATHERS_PER_SC)],
                    idx_v.at[1 - ib], sem_idx).wait()
                prime(1 - ib)

                @pl.when(sc + 2 < SUPER_PER_W)
                def _():
                    pltpu.async_copy(
                        tok_hbm.at[pl.ds(idx_rows(sc + 2), GATHERS_PER_SC)],
                        idx_v.at[ib], sem_idx)
            return carry

        lax.fori_loop(0, SUPER_PER_W, super_chunk, 0)
        for _ in range(2):
            pltpu.make_async_copy(
                sents_v.at[0],
                out_hbm.at[pl.ds(wid * BAGS_PER_W, BAGS_PER_SC)],
                sem_out).wait()

    return k(emb_weight, tok2d)


THALF = 4096
TCHUNKS = (VOCAB + 2 * THALF - 1) // (2 * THALF)
PACK_ROWS = TCHUNKS * THALF
LIN_ROWS = 2 * PACK_ROWS


def _transpose_kernel(a_ref, b_ref, out_ref):
    out_ref[:, 0:EMB] = a_ref[...].T
    out_ref[:, EMB:2 * EMB] = b_ref[...].T


def _relayout_table_tc(emb_t):
    out = pl.pallas_call(
        _transpose_kernel,
        grid=(TCHUNKS,),
        in_specs=[
            pl.BlockSpec((EMB, THALF), lambda i: (0, 2 * i)),
            pl.BlockSpec((EMB, THALF),
                         lambda i: (0, jnp.minimum(2 * i + 1, 2 * TCHUNKS - 2))),
        ],
        out_specs=pl.BlockSpec((THALF, 2 * EMB), lambda i: (i, 0)),
        out_shape=jax.ShapeDtypeStruct((PACK_ROWS, 2 * EMB), jnp.float32),
        compiler_params=pltpu.CompilerParams(
            dimension_semantics=("arbitrary",)),
    )(emb_t, emb_t)
    return out.reshape(LIN_ROWS, EMB)


def _attention_kernel(sents_ref, cls_ref, mul_ref, bias_ref, out_ref, sc_ref):
    m = jnp.dot(sents_ref[0], cls_ref[...], preferred_element_type=jnp.float32)
    sc_ref[0] = m
    for s in range(1, SENT_COUNT):
        scores = jnp.dot(sents_ref[s], cls_ref[...],
                         preferred_element_type=jnp.float32)
        sc_ref[s] = scores
        m = jnp.maximum(m, scores)
    den = jnp.zeros((BB, CPAD), jnp.float32)
    num = jnp.zeros((BB, CPAD), jnp.float32)
    for s in range(SENT_COUNT):
        e = jnp.exp(sc_ref[s] - m)
        den = den + e
        proj = jnp.dot(sents_ref[s], mul_ref[...],
                       preferred_element_type=jnp.float32)
        num = num + e * proj
    out_ref[...] = num / den + bias_ref[...]


def _attention_tc(sents_t, class_embs, multi_weight, multi_bias):
    cls_p = jnp.zeros((EMB, CPAD), jnp.float32).at[:, :CLASSES].set(class_embs.T)
    mul_p = jnp.zeros((EMB, CPAD), jnp.float32).at[:, :CLASSES].set(multi_weight.T)
    bias_p = jnp.zeros((1, CPAD), jnp.float32).at[0, :CLASSES].set(multi_bias)
    grid = BATCH // BB
    out = pl.pallas_call(
        _attention_kernel,
        grid=(grid,),
        in_specs=[
            pl.BlockSpec((SENT_COUNT, BB, EMB), lambda i: (0, i, 0)),
            pl.BlockSpec((EMB, CPAD), lambda i: (0, 0)),
            pl.BlockSpec((EMB, CPAD), lambda i: (0, 0)),
            pl.BlockSpec((1, CPAD), lambda i: (0, 0)),
        ],
        out_specs=pl.BlockSpec((BB, CPAD), lambda i: (i, 0)),
        out_shape=jax.ShapeDtypeStruct((BATCH, CPAD), jnp.float32),
        scratch_shapes=[pltpu.VMEM((SENT_COUNT, BB, CPAD), jnp.float32)],
        compiler_params=pltpu.CompilerParams(
            dimension_semantics=("arbitrary",)),
    )(sents_t, cls_p, mul_p, bias_p)
    return out[:, :CLASSES]


def kernel(tok_lists_batch, emb_weight, class_embs, multi_weight, multi_bias):
    emb_lin = _relayout_table_tc(emb_weight.T)
    assert emb_lin.shape == (LIN_ROWS, EMB)
    tok2d = tok_lists_batch.transpose(1, 0, 2).reshape(TOK_ROWS, IDX_PER_GATHER)
    sents_flat = _embbag_sc(emb_lin, tok2d)
    sents_t = sents_flat.reshape(SENT_COUNT, BATCH, EMB)
    return _attention_tc(sents_t, class_embs, multi_weight, multi_bias)


# speedup vs baseline: 1.3241x; 1.0402x over previous
import functools

import jax
import jax.numpy as jnp
from jax import lax
from jax.experimental import pallas as pl
from jax.experimental.pallas import tpu as pltpu
from jax.experimental.pallas import tpu_sc as plsc

VOCAB = 1000000
EMB = 64
CLASSES = 100
BATCH = 4096
SENT_COUNT = 26
SENT_LEN = 20

NUM_BAGS = BATCH * SENT_COUNT
NW = 32
BAGS_PER_W = NUM_BAGS // NW
BAGS_PER_GATHER = 4
IDX_PER_GATHER = BAGS_PER_GATHER * SENT_LEN
GATHERS_PER_W = BAGS_PER_W // BAGS_PER_GATHER
GATHERS_PER_SC = 104
SUPER_PER_W = GATHERS_PER_W // GATHERS_PER_SC
BAGS_PER_SC = GATHERS_PER_SC * BAGS_PER_GATHER
NBUF = 4
TOK_ROWS = NUM_BAGS * SENT_LEN // IDX_PER_GATHER

CPAD = 128
BB = 512
QS = EMB // 16


def _embbag_sc(emb_weight, tok2d):
    mesh = plsc.VectorSubcoreMesh(core_axis_name="c", subcore_axis_name="s",
                                  num_cores=2, num_subcores=16)

    @functools.partial(
        pl.kernel,
        out_type=jax.ShapeDtypeStruct((NUM_BAGS, EMB), jnp.float32),
        mesh=mesh,
        scratch_types=[
            pltpu.VMEM((2, GATHERS_PER_SC, IDX_PER_GATHER), jnp.int32),
            pltpu.VMEM((NBUF, IDX_PER_GATHER, EMB), jnp.float32),
            pltpu.VMEM((2, BAGS_PER_SC, EMB), jnp.float32),
            [pltpu.SemaphoreType.DMA] * NBUF,
            pltpu.SemaphoreType.DMA,
            pltpu.SemaphoreType.DMA,
        ],
        compiler_params=pltpu.CompilerParams(use_tc_tiling_on_sc=False),
    )
    def k(emb_hbm, tok_hbm, out_hbm, idx_v, rows_v, sents_v, sem_g,
          sem_idx, sem_out):
        wid = lax.axis_index("s") * 2 + lax.axis_index("c")

        def idx_rows(sc):
            return wid * GATHERS_PER_W + sc * GATHERS_PER_SC

        def to_packed_rows(ib, row):
            for q in range(IDX_PER_GATHER // 16):
                t = idx_v[ib, row, pl.ds(q * 16, 16)]
                r = ((t & (-2 * THALF)) | ((t & (THALF - 1)) << 1)
                     | ((t >> THBIT) & 1))
                idx_v[ib, row, pl.ds(q * 16, 16)] = r

        def start_gather(ib, j, b):
            pltpu.async_copy(emb_hbm.at[idx_v.at[ib, j]], rows_v.at[b],
                             sem_g[b])

        def wait_gather(b):
            pltpu.make_async_copy(emb_hbm.at[idx_v.at[0, 0]], rows_v.at[b],
                                  sem_g[b]).wait()

        def accumulate(ob, j, b):
            for bag in range(BAGS_PER_GATHER):
                acc = [rows_v[b, SENT_LEN * bag, pl.ds(q * 16, 16)]
                       for q in range(QS)]
                for t in range(1, SENT_LEN):
                    for q in range(QS):
                        acc[q] = acc[q] + rows_v[b, SENT_LEN * bag + t,
                                                 pl.ds(q * 16, 16)]
                row = j * BAGS_PER_GATHER + bag
                for q in range(QS):
                    sents_v[ob, row, pl.ds(q * 16, 16)] = \
                        acc[q] * (1.0 / SENT_LEN)

        def prime(ib):
            for b in range(NBUF):
                to_packed_rows(ib, b)
                start_gather(ib, b, b)

        pltpu.sync_copy(tok_hbm.at[pl.ds(idx_rows(0), GATHERS_PER_SC)],
                        idx_v.at[0])
        pltpu.async_copy(tok_hbm.at[pl.ds(idx_rows(1), GATHERS_PER_SC)],
                         idx_v.at[1], sem_idx)
        prime(0)

        def super_chunk(sc, carry):
            ib = sc & 1
            ob = sc & 1

            @pl.when(sc >= 2)
            def _():
                pltpu.make_async_copy(
                    sents_v.at[ob],
                    out_hbm.at[pl.ds(wid * BAGS_PER_W, BAGS_PER_SC)],
                    sem_out).wait()

            def quad(g, c):
                for b in range(NBUF):
                    j = g * NBUF + b
                    wait_gather(b)
                    accumulate(ob, j, b)

                    @pl.when(j + NBUF < GATHERS_PER_SC)
                    def _():
                        to_packed_rows(ib, j + NBUF)
                        start_gather(ib, j + NBUF, b)
                return c

            lax.fori_loop(0, GATHERS_PER_SC // NBUF, quad, 0)

            bag0 = wid * BAGS_PER_W + sc * BAGS_PER_SC
            pltpu.async_copy(sents_v.at[ob],
                             out_hbm.at[pl.ds(bag0, BAGS_PER_SC)], sem_out)

            @pl.when(sc + 1 < SUPER_PER_W)
            def _():
                pltpu.make_async_copy(
                    tok_hbm.at[pl.ds(idx_rows(0), GATHERS_PER_SC)],
                    idx_v.at[1 - ib], sem_idx).wait()
                prime(1 - ib)

                @pl.when(sc + 2 < SUPER_PER_W)
                def _():
                    pltpu.async_copy(
                        tok_hbm.at[pl.ds(idx_rows(sc + 2), GATHERS_PER_SC)],
                        idx_v.at[ib], sem_idx)
            return carry

        lax.fori_loop(0, SUPER_PER_W, super_chunk, 0)
        for _ in range(2):
            pltpu.make_async_copy(
                sents_v.at[0],
                out_hbm.at[pl.ds(wid * BAGS_PER_W, BAGS_PER_SC)],
                sem_out).wait()

    return k(emb_weight, tok2d)


THALF = 8192
THBIT = THALF.bit_length() - 1
TCHUNKS = (VOCAB + 2 * THALF - 1) // (2 * THALF)
PACK_ROWS = TCHUNKS * THALF
LIN_ROWS = 2 * PACK_ROWS


def _transpose_kernel(a_ref, b_ref, out_ref):
    out_ref[:, 0:EMB] = a_ref[...].T
    out_ref[:, EMB:2 * EMB] = b_ref[...].T


def _relayout_table_tc(emb_t):
    out = pl.pallas_call(
        _transpose_kernel,
        grid=(TCHUNKS,),
        in_specs=[
            pl.BlockSpec((EMB, THALF), lambda i: (0, 2 * i)),
            pl.BlockSpec((EMB, THALF),
                         lambda i: (0, jnp.minimum(2 * i + 1, 2 * TCHUNKS - 2))),
        ],
        out_specs=pl.BlockSpec((THALF, 2 * EMB), lambda i: (i, 0)),
        out_shape=jax.ShapeDtypeStruct((PACK_ROWS, 2 * EMB), jnp.float32),
        compiler_params=pltpu.CompilerParams(
            dimension_semantics=("arbitrary",)),
    )(emb_t, emb_t)
    return out.reshape(LIN_ROWS, EMB)


def _attention_kernel(sents_ref, cls_ref, mul_ref, bias_ref, out_ref, sc_ref):
    m = jnp.dot(sents_ref[0], cls_ref[...], preferred_element_type=jnp.float32)
    sc_ref[0] = m
    for s in range(1, SENT_COUNT):
        scores = jnp.dot(sents_ref[s], cls_ref[...],
                         preferred_element_type=jnp.float32)
        sc_ref[s] = scores
        m = jnp.maximum(m, scores)
    den = jnp.zeros((BB, CPAD), jnp.float32)
    num = jnp.zeros((BB, CPAD), jnp.float32)
    for s in range(SENT_COUNT):
        e = jnp.exp(sc_ref[s] - m)
        den = den + e
        proj = jnp.dot(sents_ref[s], mul_ref[...],
                       preferred_element_type=jnp.float32)
        num = num + e * proj
    out_ref[...] = num / den + bias_ref[...]


def _attention_tc(sents_t, class_embs, multi_weight, multi_bias):
    cls_p = jnp.zeros((EMB, CPAD), jnp.float32).at[:, :CLASSES].set(class_embs.T)
    mul_p = jnp.zeros((EMB, CPAD), jnp.float32).at[:, :CLASSES].set(multi_weight.T)
    bias_p = jnp.zeros((1, CPAD), jnp.float32).at[0, :CLASSES].set(multi_bias)
    grid = BATCH // BB
    out = pl.pallas_call(
        _attention_kernel,
        grid=(grid,),
        in_specs=[
            pl.BlockSpec((SENT_COUNT, BB, EMB), lambda i: (0, i, 0)),
            pl.BlockSpec((EMB, CPAD), lambda i: (0, 0)),
            pl.BlockSpec((EMB, CPAD), lambda i: (0, 0)),
            pl.BlockSpec((1, CPAD), lambda i: (0, 0)),
        ],
        out_specs=pl.BlockSpec((BB, CPAD), lambda i: (i, 0)),
        out_shape=jax.ShapeDtypeStruct((BATCH, CPAD), jnp.float32),
        scratch_shapes=[pltpu.VMEM((SENT_COUNT, BB, CPAD), jnp.float32)],
        compiler_params=pltpu.CompilerParams(
            dimension_semantics=("arbitrary",)),
    )(sents_t, cls_p, mul_p, bias_p)
    return out[:, :CLASSES]


def kernel(tok_lists_batch, emb_weight, class_embs, multi_weight, multi_bias):
    emb_lin = _relayout_table_tc(emb_weight.T)
    assert emb_lin.shape == (LIN_ROWS, EMB)
    tok2d = tok_lists_batch.transpose(1, 0, 2).reshape(TOK_ROWS, IDX_PER_GATHER)
    sents_flat = _embbag_sc(emb_lin, tok2d)
    sents_t = sents_flat.reshape(SENT_COUNT, BATCH, EMB)
    return _attention_tc(sents_t, class_embs, multi_weight, multi_bias)


# speedup vs baseline: 1.3508x; 1.0201x over previous
import functools

import jax
import jax.numpy as jnp
from jax import lax
from jax.experimental import pallas as pl
from jax.experimental.pallas import tpu as pltpu
from jax.experimental.pallas import tpu_sc as plsc

VOCAB = 1000000
EMB = 64
CLASSES = 100
BATCH = 4096
SENT_COUNT = 26
SENT_LEN = 20

NUM_BAGS = BATCH * SENT_COUNT
NW = 32
BAGS_PER_W = NUM_BAGS // NW
BAGS_PER_GATHER = 4
IDX_PER_GATHER = BAGS_PER_GATHER * SENT_LEN
GATHERS_PER_W = BAGS_PER_W // BAGS_PER_GATHER
GATHERS_PER_SC = 104
SUPER_PER_W = GATHERS_PER_W // GATHERS_PER_SC
BAGS_PER_SC = GATHERS_PER_SC * BAGS_PER_GATHER
NBUF = 4
TOK_ROWS = NUM_BAGS * SENT_LEN // IDX_PER_GATHER

CPAD = 128
BB = 512
QS = EMB // 16


def _embbag_sc(emb_weight, tok2d):
    mesh = plsc.VectorSubcoreMesh(core_axis_name="c", subcore_axis_name="s",
                                  num_cores=2, num_subcores=16)

    @functools.partial(
        pl.kernel,
        out_type=jax.ShapeDtypeStruct((NUM_BAGS, EMB), jnp.float32),
        mesh=mesh,
        scratch_types=[
            pltpu.VMEM((2, GATHERS_PER_SC, IDX_PER_GATHER), jnp.int32),
            pltpu.VMEM((NBUF, IDX_PER_GATHER, EMB), jnp.float32),
            pltpu.VMEM((2, BAGS_PER_SC, EMB), jnp.float32),
            [pltpu.SemaphoreType.DMA] * NBUF,
            pltpu.SemaphoreType.DMA,
            pltpu.SemaphoreType.DMA,
        ],
        compiler_params=pltpu.CompilerParams(use_tc_tiling_on_sc=False),
    )
    def k(emb_hbm, tok_hbm, out_hbm, idx_v, rows_v, sents_v, sem_g,
          sem_idx, sem_out):
        wid = lax.axis_index("s") * 2 + lax.axis_index("c")

        def idx_rows(sc):
            return wid * GATHERS_PER_W + sc * GATHERS_PER_SC

        def to_packed_rows(ib, row):
            for q in range(IDX_PER_GATHER // 16):
                t = idx_v[ib, row, pl.ds(q * 16, 16)]
                r = ((t & (-2 * THALF)) | ((t & (THALF - 1)) << 1)
                     | ((t >> THBIT) & 1))
                idx_v[ib, row, pl.ds(q * 16, 16)] = r

        def start_gather(ib, j, b):
            pltpu.async_copy(emb_hbm.at[idx_v.at[ib, j]], rows_v.at[b],
                             sem_g[b])

        def wait_gather(b):
            pltpu.make_async_copy(emb_hbm.at[idx_v.at[0, 0]], rows_v.at[b],
                                  sem_g[b]).wait()

        def accumulate(ob, j, b):
            for bag in range(BAGS_PER_GATHER):
                acc = [rows_v[b, SENT_LEN * bag, pl.ds(q * 16, 16)]
                       for q in range(QS)]
                for t in range(1, SENT_LEN):
                    for q in range(QS):
                        acc[q] = acc[q] + rows_v[b, SENT_LEN * bag + t,
                                                 pl.ds(q * 16, 16)]
                row = j * BAGS_PER_GATHER + bag
                for q in range(QS):
                    sents_v[ob, row, pl.ds(q * 16, 16)] = \
                        acc[q] * (1.0 / SENT_LEN)

        def prime(ib):
            for b in range(NBUF):
                to_packed_rows(ib, b)
                start_gather(ib, b, b)

        pltpu.sync_copy(tok_hbm.at[pl.ds(idx_rows(0), GATHERS_PER_SC)],
                        idx_v.at[0])
        pltpu.async_copy(tok_hbm.at[pl.ds(idx_rows(1), GATHERS_PER_SC)],
                         idx_v.at[1], sem_idx)
        prime(0)

        def super_chunk(sc, carry):
            ib = sc & 1
            ob = sc & 1

            @pl.when(sc >= 2)
            def _():
                pltpu.make_async_copy(
                    sents_v.at[ob],
                    out_hbm.at[pl.ds(wid * BAGS_PER_W, BAGS_PER_SC)],
                    sem_out).wait()

            def quad(g, c):
                for b in range(NBUF):
                    j = g * NBUF + b
                    wait_gather(b)
                    accumulate(ob, j, b)

                    @pl.when(j + NBUF < GATHERS_PER_SC)
                    def _():
                        to_packed_rows(ib, j + NBUF)
                        start_gather(ib, j + NBUF, b)
                return c

            lax.fori_loop(0, GATHERS_PER_SC // NBUF, quad, 0)

            bag0 = wid * BAGS_PER_W + sc * BAGS_PER_SC
            pltpu.async_copy(sents_v.at[ob],
                             out_hbm.at[pl.ds(bag0, BAGS_PER_SC)], sem_out)

            @pl.when(sc + 1 < SUPER_PER_W)
            def _():
                pltpu.make_async_copy(
                    tok_hbm.at[pl.ds(idx_rows(0), GATHERS_PER_SC)],
                    idx_v.at[1 - ib], sem_idx).wait()
                prime(1 - ib)

                @pl.when(sc + 2 < SUPER_PER_W)
                def _():
                    pltpu.async_copy(
                        tok_hbm.at[pl.ds(idx_rows(sc + 2), GATHERS_PER_SC)],
                        idx_v.at[ib], sem_idx)
            return carry

        lax.fori_loop(0, SUPER_PER_W, super_chunk, 0)
        for _ in range(2):
            pltpu.make_async_copy(
                sents_v.at[0],
                out_hbm.at[pl.ds(wid * BAGS_PER_W, BAGS_PER_SC)],
                sem_out).wait()

    return k(emb_weight, tok2d)


THALF = 16384
THBIT = THALF.bit_length() - 1
TCHUNKS = (VOCAB + 2 * THALF - 1) // (2 * THALF)
PACK_ROWS = TCHUNKS * THALF
LIN_ROWS = 2 * PACK_ROWS


def _transpose_kernel(a_ref, b_ref, out_ref):
    out_ref[:, 0:EMB] = a_ref[...].T
    out_ref[:, EMB:2 * EMB] = b_ref[...].T


def _relayout_table_tc(emb_t):
    out = pl.pallas_call(
        _transpose_kernel,
        grid=(TCHUNKS,),
        in_specs=[
            pl.BlockSpec((EMB, THALF), lambda i: (0, 2 * i)),
            pl.BlockSpec((EMB, THALF),
                         lambda i: (0, jnp.minimum(
                             2 * i + 1, (VOCAB + THALF - 1) // THALF - 1))),
        ],
        out_specs=pl.BlockSpec((THALF, 2 * EMB), lambda i: (i, 0)),
        out_shape=jax.ShapeDtypeStruct((PACK_ROWS, 2 * EMB), jnp.float32),
        compiler_params=pltpu.CompilerParams(
            dimension_semantics=("arbitrary",)),
    )(emb_t, emb_t)
    return out.reshape(LIN_ROWS, EMB)


def _attention_kernel(sents_ref, cls_ref, mul_ref, bias_ref, out_ref, sc_ref):
    m = jnp.dot(sents_ref[0], cls_ref[...], preferred_element_type=jnp.float32)
    sc_ref[0] = m
    for s in range(1, SENT_COUNT):
        scores = jnp.dot(sents_ref[s], cls_ref[...],
                         preferred_element_type=jnp.float32)
        sc_ref[s] = scores
        m = jnp.maximum(m, scores)
    den = jnp.zeros((BB, CPAD), jnp.float32)
    num = jnp.zeros((BB, CPAD), jnp.float32)
    for s in range(SENT_COUNT):
        e = jnp.exp(sc_ref[s] - m)
        den = den + e
        proj = jnp.dot(sents_ref[s], mul_ref[...],
                       preferred_element_type=jnp.float32)
        num = num + e * proj
    out_ref[...] = num / den + bias_ref[...]


def _attention_tc(sents_t, class_embs, multi_weight, multi_bias):
    cls_p = jnp.zeros((EMB, CPAD), jnp.float32).at[:, :CLASSES].set(class_embs.T)
    mul_p = jnp.zeros((EMB, CPAD), jnp.float32).at[:, :CLASSES].set(multi_weight.T)
    bias_p = jnp.zeros((1, CPAD), jnp.float32).at[0, :CLASSES].set(multi_bias)
    grid = BATCH // BB
    out = pl.pallas_call(
        _attention_kernel,
        grid=(grid,),
        in_specs=[
            pl.BlockSpec((SENT_COUNT, BB, EMB), lambda i: (0, i, 0)),
            pl.BlockSpec((EMB, CPAD), lambda i: (0, 0)),
            pl.BlockSpec((EMB, CPAD), lambda i: (0, 0)),
            pl.BlockSpec((1, CPAD), lambda i: (0, 0)),
        ],
        out_specs=pl.BlockSpec((BB, CPAD), lambda i: (i, 0)),
        out_shape=jax.ShapeDtypeStruct((BATCH, CPAD), jnp.float32),
        scratch_shapes=[pltpu.VMEM((SENT_COUNT, BB, CPAD), jnp.float32)],
        compiler_params=pltpu.CompilerParams(
            dimension_semantics=("arbitrary",)),
    )(sents_t, cls_p, mul_p, bias_p)
    return out[:, :CLASSES]


def kernel(tok_lists_batch, emb_weight, class_embs, multi_weight, multi_bias):
    emb_lin = _relayout_table_tc(emb_weight.T)
    assert emb_lin.shape == (LIN_ROWS, EMB)
    tok2d = tok_lists_batch.transpose(1, 0, 2).reshape(TOK_ROWS, IDX_PER_GATHER)
    sents_flat = _embbag_sc(emb_lin, tok2d)
    sents_t = sents_flat.reshape(SENT_COUNT, BATCH, EMB)
    return _attention_tc(sents_t, class_embs, multi_weight, multi_bias)


# speedup vs baseline: 1.4232x; 1.0536x over previous
import functools

import jax
import jax.numpy as jnp
from jax import lax
from jax.experimental import pallas as pl
from jax.experimental.pallas import tpu as pltpu
from jax.experimental.pallas import tpu_sc as plsc

VOCAB = 1000000
EMB = 64
CLASSES = 100
BATCH = 4096
SENT_COUNT = 26
SENT_LEN = 20

NUM_BAGS = BATCH * SENT_COUNT
NW = 32
BAGS_PER_W = NUM_BAGS // NW
BAGS_PER_GATHER = 4
IDX_PER_GATHER = BAGS_PER_GATHER * SENT_LEN
GATHERS_PER_W = BAGS_PER_W // BAGS_PER_GATHER
GATHERS_PER_SC = 104
SUPER_PER_W = GATHERS_PER_W // GATHERS_PER_SC
BAGS_PER_SC = GATHERS_PER_SC * BAGS_PER_GATHER
NBUF = 4
TOK_ROWS = NUM_BAGS * SENT_LEN // IDX_PER_GATHER

CPAD = 128
PB = 256
QS = EMB // 16


def _embbag_sc(emb_weight, tok2d):
    mesh = plsc.VectorSubcoreMesh(core_axis_name="c", subcore_axis_name="s",
                                  num_cores=2, num_subcores=16)

    @functools.partial(
        pl.kernel,
        out_type=jax.ShapeDtypeStruct((NUM_BAGS, EMB), jnp.float32),
        mesh=mesh,
        scratch_types=[
            pltpu.VMEM((2, GATHERS_PER_SC, IDX_PER_GATHER), jnp.int32),
            pltpu.VMEM((NBUF, IDX_PER_GATHER, EMB), jnp.float32),
            pltpu.VMEM((2, BAGS_PER_SC, EMB), jnp.float32),
            [pltpu.SemaphoreType.DMA] * NBUF,
            pltpu.SemaphoreType.DMA,
            pltpu.SemaphoreType.DMA,
        ],
        compiler_params=pltpu.CompilerParams(use_tc_tiling_on_sc=False),
    )
    def k(emb_hbm, tok_hbm, out_hbm, idx_v, rows_v, sents_v, sem_g,
          sem_idx, sem_out):
        wid = lax.axis_index("s") * 2 + lax.axis_index("c")

        def idx_rows(sc):
            return wid * GATHERS_PER_W + sc * GATHERS_PER_SC

        def to_packed_rows(ib, row):
            for q in range(IDX_PER_GATHER // 16):
                t = idx_v[ib, row, pl.ds(q * 16, 16)]
                r = ((t & (-2 * THALF)) | ((t & (THALF - 1)) << 1)
                     | ((t >> THBIT) & 1))
                idx_v[ib, row, pl.ds(q * 16, 16)] = r

        def start_gather(ib, j, b):
            pltpu.async_copy(emb_hbm.at[idx_v.at[ib, j]], rows_v.at[b],
                             sem_g[b])

        def wait_gather(b):
            pltpu.make_async_copy(emb_hbm.at[idx_v.at[0, 0]], rows_v.at[b],
                                  sem_g[b]).wait()

        def accumulate(ob, j, b):
            for bag in range(BAGS_PER_GATHER):
                acc = [rows_v[b, SENT_LEN * bag, pl.ds(q * 16, 16)]
                       for q in range(QS)]
                for t in range(1, SENT_LEN):
                    for q in range(QS):
                        acc[q] = acc[q] + rows_v[b, SENT_LEN * bag + t,
                                                 pl.ds(q * 16, 16)]
                row = j * BAGS_PER_GATHER + bag
                for q in range(QS):
                    sents_v[ob, row, pl.ds(q * 16, 16)] = \
                        acc[q] * (1.0 / SENT_LEN)

        def prime(ib):
            for b in range(NBUF):
                to_packed_rows(ib, b)
                start_gather(ib, b, b)

        pltpu.sync_copy(tok_hbm.at[pl.ds(idx_rows(0), GATHERS_PER_SC)],
                        idx_v.at[0])
        pltpu.async_copy(tok_hbm.at[pl.ds(idx_rows(1), GATHERS_PER_SC)],
                         idx_v.at[1], sem_idx)
        prime(0)

        def super_chunk(sc, carry):
            ib = sc & 1
            ob = sc & 1

            @pl.when(sc >= 2)
            def _():
                pltpu.make_async_copy(
                    sents_v.at[ob],
                    out_hbm.at[pl.ds(wid * BAGS_PER_W, BAGS_PER_SC)],
                    sem_out).wait()

            def quad(g, c):
                for b in range(NBUF):
                    j = g * NBUF + b
                    wait_gather(b)
                    accumulate(ob, j, b)

                    @pl.when(j + NBUF < GATHERS_PER_SC)
                    def _():
                        to_packed_rows(ib, j + NBUF)
                        start_gather(ib, j + NBUF, b)
                return c

            lax.fori_loop(0, GATHERS_PER_SC // NBUF, quad, 0)

            bag0 = wid * BAGS_PER_W + sc * BAGS_PER_SC
            pltpu.async_copy(sents_v.at[ob],
                             out_hbm.at[pl.ds(bag0, BAGS_PER_SC)], sem_out)

            @pl.when(sc + 1 < SUPER_PER_W)
            def _():
                pltpu.make_async_copy(
                    tok_hbm.at[pl.ds(idx_rows(0), GATHERS_PER_SC)],
                    idx_v.at[1 - ib], sem_idx).wait()
                prime(1 - ib)

                @pl.when(sc + 2 < SUPER_PER_W)
                def _():
                    pltpu.async_copy(
                        tok_hbm.at[pl.ds(idx_rows(sc + 2), GATHERS_PER_SC)],
                        idx_v.at[ib], sem_idx)
            return carry

        lax.fori_loop(0, SUPER_PER_W, super_chunk, 0)
        for _ in range(2):
            pltpu.make_async_copy(
                sents_v.at[0],
                out_hbm.at[pl.ds(wid * BAGS_PER_W, BAGS_PER_SC)],
                sem_out).wait()

    return k(emb_weight, tok2d)


THALF = 16384
THBIT = THALF.bit_length() - 1
TCHUNKS = (VOCAB + 2 * THALF - 1) // (2 * THALF)
PACK_ROWS = TCHUNKS * THALF
LIN_ROWS = 2 * PACK_ROWS


def _transpose_kernel(a_ref, b_ref, out_ref):
    out_ref[:, 0:EMB] = a_ref[...].T
    out_ref[:, EMB:2 * EMB] = b_ref[...].T


def _relayout_table_tc(emb_t):
    out = pl.pallas_call(
        _transpose_kernel,
        grid=(TCHUNKS,),
        in_specs=[
            pl.BlockSpec((EMB, THALF), lambda i: (0, 2 * i)),
            pl.BlockSpec((EMB, THALF),
                         lambda i: (0, jnp.minimum(
                             2 * i + 1, (VOCAB + THALF - 1) // THALF - 1))),
        ],
        out_specs=pl.BlockSpec((THALF, 2 * EMB), lambda i: (i, 0)),
        out_shape=jax.ShapeDtypeStruct((PACK_ROWS, 2 * EMB), jnp.float32),
        compiler_params=pltpu.CompilerParams(
            dimension_semantics=("arbitrary",)),
    )(emb_t, emb_t)
    return out.reshape(LIN_ROWS, EMB)


def _attention_kernel(sents_ref, cls_ref, mul_ref, bias_ref, out_ref,
                      sce_ref, sco_ref):
    halves = []
    for lo in (0, EMB):
        sc_ref = sce_ref if lo == 0 else sco_ref
        x0 = sents_ref[0][:, lo:lo + EMB]
        m = jnp.dot(x0, cls_ref[...], preferred_element_type=jnp.float32)
        sc_ref[0] = m
        for s in range(1, SENT_COUNT):
            scores = jnp.dot(sents_ref[s][:, lo:lo + EMB], cls_ref[...],
                             preferred_element_type=jnp.float32)
            sc_ref[s] = scores
            m = jnp.maximum(m, scores)
        den = jnp.zeros((PB, CPAD), jnp.float32)
        num = jnp.zeros((PB, CPAD), jnp.float32)
        for s in range(SENT_COUNT):
            e = jnp.exp(sc_ref[s] - m)
            den = den + e
            proj = jnp.dot(sents_ref[s][:, lo:lo + EMB], mul_ref[...],
                           preferred_element_type=jnp.float32)
            num = num + e * proj
        halves.append(num / den + bias_ref[...])
    out_ref[...] = jnp.concatenate(halves, axis=1)


def _attention_tc(sents_packed, class_embs, multi_weight, multi_bias):
    cls_p = jnp.zeros((EMB, CPAD), jnp.float32).at[:, :CLASSES].set(class_embs.T)
    mul_p = jnp.zeros((EMB, CPAD), jnp.float32).at[:, :CLASSES].set(multi_weight.T)
    bias_p = jnp.zeros((1, CPAD), jnp.float32).at[0, :CLASSES].set(multi_bias)
    grid = (BATCH // 2) // PB
    out = pl.pallas_call(
        _attention_kernel,
        grid=(grid,),
        in_specs=[
            pl.BlockSpec((SENT_COUNT, PB, 2 * EMB), lambda i: (0, i, 0)),
            pl.BlockSpec((EMB, CPAD), lambda i: (0, 0)),
            pl.BlockSpec((EMB, CPAD), lambda i: (0, 0)),
            pl.BlockSpec((1, CPAD), lambda i: (0, 0)),
        ],
        out_specs=pl.BlockSpec((PB, 2 * CPAD), lambda i: (i, 0)),
        out_shape=jax.ShapeDtypeStruct((BATCH // 2, 2 * CPAD), jnp.float32),
        scratch_shapes=[pltpu.VMEM((SENT_COUNT, PB, CPAD), jnp.float32),
                        pltpu.VMEM((SENT_COUNT, PB, CPAD), jnp.float32)],
        compiler_params=pltpu.CompilerParams(
            dimension_semantics=("arbitrary",)),
    )(sents_packed, cls_p, mul_p, bias_p)
    return out.reshape(BATCH, CPAD)[:, :CLASSES]


def kernel(tok_lists_batch, emb_weight, class_embs, multi_weight, multi_bias):
    emb_lin = _relayout_table_tc(emb_weight.T)
    assert emb_lin.shape == (LIN_ROWS, EMB)
    tok2d = tok_lists_batch.transpose(1, 0, 2).reshape(TOK_ROWS, IDX_PER_GATHER)
    sents_flat = _embbag_sc(emb_lin, tok2d)
    sents_packed = sents_flat.reshape(SENT_COUNT, BATCH // 2, 2 * EMB)
    return _attention_tc(sents_packed, class_embs, multi_weight, multi_bias)


# speedup vs baseline: 1.9226x; 1.3509x over previous
import functools

import jax
import jax.numpy as jnp
from jax import lax
from jax.experimental import pallas as pl
from jax.experimental.pallas import tpu as pltpu
from jax.experimental.pallas import tpu_sc as plsc

VOCAB = 1000000
EMB = 64
CLASSES = 100
BATCH = 4096
SENT_COUNT = 26
SENT_LEN = 20

NUM_BAGS = BATCH * SENT_COUNT
NW = 32
BAGS_PER_W = NUM_BAGS // NW
BAGS_PER_GATHER = 4
IDX_PER_GATHER = BAGS_PER_GATHER * SENT_LEN
GATHERS_PER_W = BAGS_PER_W // BAGS_PER_GATHER
GATHERS_PER_SC = 104
SUPER_PER_W = GATHERS_PER_W // GATHERS_PER_SC
BAGS_PER_SC = GATHERS_PER_SC * BAGS_PER_GATHER
NBUF = 4
TOK_ROWS = NUM_BAGS * SENT_LEN // IDX_PER_GATHER

CPAD = 128
PB = 256
QS = EMB // 16


def _embbag_sc(emb_weight, tok2d):
    mesh = plsc.VectorSubcoreMesh(core_axis_name="c", subcore_axis_name="s",
                                  num_cores=2, num_subcores=16)

    @functools.partial(
        pl.kernel,
        out_type=jax.ShapeDtypeStruct((NUM_BAGS, EMB), jnp.float32),
        mesh=mesh,
        scratch_types=[
            pltpu.VMEM((2, GATHERS_PER_SC, IDX_PER_GATHER), jnp.int32),
            pltpu.VMEM((NBUF, IDX_PER_GATHER, EMB), jnp.float32),
            pltpu.VMEM((2, BAGS_PER_SC, EMB), jnp.float32),
            [pltpu.SemaphoreType.DMA] * NBUF,
            pltpu.SemaphoreType.DMA,
            pltpu.SemaphoreType.DMA,
        ],
        compiler_params=pltpu.CompilerParams(use_tc_tiling_on_sc=False),
    )
    def k(emb_hbm, tok_hbm, out_hbm, idx_v, rows_v, sents_v, sem_g,
          sem_idx, sem_out):
        wid = lax.axis_index("s") * 2 + lax.axis_index("c")

        def idx_rows(sc):
            return wid * GATHERS_PER_W + sc * GATHERS_PER_SC

        def start_gather(ib, j, b):
            pltpu.async_copy(emb_hbm.at[idx_v.at[ib, j]], rows_v.at[b],
                             sem_g[b])

        def wait_gather(b):
            pltpu.make_async_copy(emb_hbm.at[idx_v.at[0, 0]], rows_v.at[b],
                                  sem_g[b]).wait()

        def accumulate(ob, j, b):
            for bag in range(BAGS_PER_GATHER):
                acc = [rows_v[b, SENT_LEN * bag, pl.ds(q * 16, 16)]
                       for q in range(QS)]
                for t in range(1, SENT_LEN):
                    for q in range(QS):
                        acc[q] = acc[q] + rows_v[b, SENT_LEN * bag + t,
                                                 pl.ds(q * 16, 16)]
                row = j * BAGS_PER_GATHER + bag
                for q in range(QS):
                    sents_v[ob, row, pl.ds(q * 16, 16)] = \
                        acc[q] * (1.0 / SENT_LEN)

        def prime(ib):
            for b in range(NBUF):
                start_gather(ib, b, b)

        pltpu.sync_copy(tok_hbm.at[pl.ds(idx_rows(0), GATHERS_PER_SC)],
                        idx_v.at[0])
        pltpu.async_copy(tok_hbm.at[pl.ds(idx_rows(1), GATHERS_PER_SC)],
                         idx_v.at[1], sem_idx)
        prime(0)

        def super_chunk(sc, carry):
            ib = sc & 1
            ob = sc & 1

            @pl.when(sc >= 2)
            def _():
                pltpu.make_async_copy(
                    sents_v.at[ob],
                    out_hbm.at[pl.ds(wid * BAGS_PER_W, BAGS_PER_SC)],
                    sem_out).wait()

            def quad(g, c):
                for b in range(NBUF):
                    j = g * NBUF + b
                    wait_gather(b)
                    accumulate(ob, j, b)

                    @pl.when(j + NBUF < GATHERS_PER_SC)
                    def _():
                        start_gather(ib, j + NBUF, b)
                return c

            lax.fori_loop(0, GATHERS_PER_SC // NBUF, quad, 0)

            bag0 = wid * BAGS_PER_W + sc * BAGS_PER_SC
            pltpu.async_copy(sents_v.at[ob],
                             out_hbm.at[pl.ds(bag0, BAGS_PER_SC)], sem_out)

            @pl.when(sc + 1 < SUPER_PER_W)
            def _():
                pltpu.make_async_copy(
                    tok_hbm.at[pl.ds(idx_rows(0), GATHERS_PER_SC)],
                    idx_v.at[1 - ib], sem_idx).wait()
                prime(1 - ib)

                @pl.when(sc + 2 < SUPER_PER_W)
                def _():
                    pltpu.async_copy(
                        tok_hbm.at[pl.ds(idx_rows(sc + 2), GATHERS_PER_SC)],
                        idx_v.at[ib], sem_idx)
            return carry

        lax.fori_loop(0, SUPER_PER_W, super_chunk, 0)
        for _ in range(2):
            pltpu.make_async_copy(
                sents_v.at[0],
                out_hbm.at[pl.ds(wid * BAGS_PER_W, BAGS_PER_SC)],
                sem_out).wait()

    return k(emb_weight, tok2d)


THALF = 16384
THBIT = THALF.bit_length() - 1
TCHUNKS = (VOCAB + 2 * THALF - 1) // (2 * THALF)
PACK_ROWS = TCHUNKS * THALF
LIN_ROWS = 2 * PACK_ROWS


def _transpose_kernel(a_ref, b_ref, out_ref):
    out_ref[:, 0:EMB] = a_ref[...].T
    out_ref[:, EMB:2 * EMB] = b_ref[...].T


def _relayout_table_tc(emb_t):
    out = pl.pallas_call(
        _transpose_kernel,
        grid=(TCHUNKS,),
        in_specs=[
            pl.BlockSpec((EMB, THALF), lambda i: (0, 2 * i)),
            pl.BlockSpec((EMB, THALF),
                         lambda i: (0, jnp.minimum(
                             2 * i + 1, (VOCAB + THALF - 1) // THALF - 1))),
        ],
        out_specs=pl.BlockSpec((THALF, 2 * EMB), lambda i: (i, 0)),
        out_shape=jax.ShapeDtypeStruct((PACK_ROWS, 2 * EMB), jnp.float32),
        compiler_params=pltpu.CompilerParams(
            dimension_semantics=("arbitrary",)),
    )(emb_t, emb_t)
    return out.reshape(LIN_ROWS, EMB)


def _attention_kernel(sents_ref, cls_ref, mul_ref, bias_ref, out_ref,
                      sce_ref, sco_ref):
    halves = []
    for lo in (0, EMB):
        sc_ref = sce_ref if lo == 0 else sco_ref
        x0 = sents_ref[0][:, lo:lo + EMB]
        m = jnp.dot(x0, cls_ref[...], preferred_element_type=jnp.float32)
        sc_ref[0] = m
        for s in range(1, SENT_COUNT):
            scores = jnp.dot(sents_ref[s][:, lo:lo + EMB], cls_ref[...],
                             preferred_element_type=jnp.float32)
            sc_ref[s] = scores
            m = jnp.maximum(m, scores)
        den = jnp.zeros((PB, CPAD), jnp.float32)
        num = jnp.zeros((PB, CPAD), jnp.float32)
        for s in range(SENT_COUNT):
            e = jnp.exp(sc_ref[s] - m)
            den = den + e
            proj = jnp.dot(sents_ref[s][:, lo:lo + EMB], mul_ref[...],
                           preferred_element_type=jnp.float32)
            num = num + e * proj
        halves.append(num / den + bias_ref[...])
    out_ref[...] = jnp.concatenate(halves, axis=1)


def _attention_tc(sents_packed, class_embs, multi_weight, multi_bias):
    cls_p = jnp.zeros((EMB, CPAD), jnp.float32).at[:, :CLASSES].set(class_embs.T)
    mul_p = jnp.zeros((EMB, CPAD), jnp.float32).at[:, :CLASSES].set(multi_weight.T)
    bias_p = jnp.zeros((1, CPAD), jnp.float32).at[0, :CLASSES].set(multi_bias)
    grid = (BATCH // 2) // PB
    out = pl.pallas_call(
        _attention_kernel,
        grid=(grid,),
        in_specs=[
            pl.BlockSpec((SENT_COUNT, PB, 2 * EMB), lambda i: (0, i, 0)),
            pl.BlockSpec((EMB, CPAD), lambda i: (0, 0)),
            pl.BlockSpec((EMB, CPAD), lambda i: (0, 0)),
            pl.BlockSpec((1, CPAD), lambda i: (0, 0)),
        ],
        out_specs=pl.BlockSpec((PB, 2 * CPAD), lambda i: (i, 0)),
        out_shape=jax.ShapeDtypeStruct((BATCH // 2, 2 * CPAD), jnp.float32),
        scratch_shapes=[pltpu.VMEM((SENT_COUNT, PB, CPAD), jnp.float32),
                        pltpu.VMEM((SENT_COUNT, PB, CPAD), jnp.float32)],
        compiler_params=pltpu.CompilerParams(
            dimension_semantics=("arbitrary",)),
    )(sents_packed, cls_p, mul_p, bias_p)
    return out.reshape(BATCH, CPAD)[:, :CLASSES]


def kernel(tok_lists_batch, emb_weight, class_embs, multi_weight, multi_bias):
    emb_lin = _relayout_table_tc(emb_weight.T)
    assert emb_lin.shape == (LIN_ROWS, EMB)
    t = tok_lists_batch.transpose(1, 0, 2).reshape(TOK_ROWS, IDX_PER_GATHER)
    tok2d = ((t & (-2 * THALF)) | ((t & (THALF - 1)) << 1)
             | ((t >> THBIT) & 1))
    sents_flat = _embbag_sc(emb_lin, tok2d)
    sents_packed = sents_flat.reshape(SENT_COUNT, BATCH // 2, 2 * EMB)
    return _attention_tc(sents_packed, class_embs, multi_weight, multi_bias)
